# baseline (speedup 1.0000x reference)
import functools

import jax
import jax.numpy as jnp
from jax import lax
from jax.experimental import pallas as pl
from jax.experimental.pallas import tpu as pltpu
from jax.experimental.pallas import tpu_sc as plsc

_N_CODES = 8192
_DIM = 256
_TOKENS = 8192
_TT = 512
_CT = 8192
_I32_MAX = 2**31 - 1


def _argmin_body(en_ref, iotaf_ref, z_ref, emb_ref, idx_ref, z2_ref):
    j = pl.program_id(1)
    zt = jnp.transpose(z_ref[0, :, pl.ds(j * _TT, _TT)], (1, 0))
    z2_ref[...] = zt * -2.0
    zn = jnp.sum(zt ** 2, axis=1, keepdims=True)
    s2 = lax.dot_general(
        z2_ref[...], emb_ref[...],
        dimension_numbers=(((1,), (1,)), ((), ())),
        preferred_element_type=jnp.float32,
    )
    d = (zn + en_ref[...]) + s2
    local_min = jnp.min(d, axis=1, keepdims=True)
    idx_f = jnp.min(
        jnp.where(d == local_min, iotaf_ref[...], jnp.float32(3.0e38)),
        axis=1, keepdims=True,
    )
    idx_ref[...] = idx_f.astype(jnp.int32)


def _fused_argmin(en, iotaf, z3, emb):
    npos = z3.shape[2]
    grid = (z3.shape[0], npos // _TT)
    return pl.pallas_call(
        _argmin_body,
        grid=grid,
        in_specs=[
            pl.BlockSpec((1, _N_CODES), lambda b, j: (0, 0)),
            pl.BlockSpec((1, _N_CODES), lambda b, j: (0, 0)),
            pl.BlockSpec((1, _DIM, npos), lambda b, j: (b, 0, 0)),
            pl.BlockSpec((_N_CODES, _DIM), lambda b, j: (0, 0)),
        ],
        out_specs=pl.BlockSpec((_TT, 1), lambda b, j: (b * (npos // _TT) + j, 0)),
        out_shape=jax.ShapeDtypeStruct((_TOKENS, 1), jnp.int32),
        scratch_shapes=[
            pltpu.VMEM((_TT, _DIM), jnp.float32),
        ],
    )(en, iotaf, z3, emb)


def _sc_gather(embedding, idx):
    info = plsc.get_sparse_core_info()
    nw = info.num_cores * info.num_subcores
    b_per_w = _TOKENS // nw
    mesh = plsc.VectorSubcoreMesh(core_axis_name="c", subcore_axis_name="s")

    @functools.partial(
        pl.kernel,
        mesh=mesh,
        out_type=jax.ShapeDtypeStruct((_TOKENS, _DIM), jnp.float32),
        scratch_types=[
            pltpu.VMEM((b_per_w,), jnp.int32),
            pltpu.VMEM((b_per_w, _DIM), jnp.float32),
            pltpu.SemaphoreType.DMA,
        ],
    )
    def k(table_hbm, idx_hbm, out_hbm, idx_v, rows_v, sem):
        wid = lax.axis_index("s") * info.num_cores + lax.axis_index("c")
        base = wid * b_per_w
        pltpu.sync_copy(idx_hbm.at[pl.ds(base, b_per_w)], idx_v)
        pltpu.async_copy(table_hbm.at[idx_v], rows_v, sem).wait()
        pltpu.sync_copy(rows_v, out_hbm.at[pl.ds(base, b_per_w)])

    return k(embedding, idx)


def kernel(z, embedding):
    b, c, h, w = z.shape
    z3 = z.reshape(b, c, h * w)
    en = jnp.sum(embedding**2, axis=1)[None, :]
    iotaf = lax.iota(jnp.float32, _N_CODES)[None, :]
    idx2d = _fused_argmin(en, iotaf, z3, embedding)
    min_indices = idx2d.reshape(-1)
    z_q_flat = _sc_gather(embedding, min_indices)
    z_q_bchw = jnp.transpose(z_q_flat.reshape(b, h, w, c), (0, 3, 1, 2))
    z_q_out = z + lax.stop_gradient(z_q_bchw - z)
    loss = jnp.array(0.0, dtype=jnp.float32)
    return (z_q_out, loss, (min_indices,))


# speedup vs baseline: 1.0977x; 1.0977x over previous
import functools

import jax
import jax.numpy as jnp
from jax import lax
from jax.experimental import pallas as pl
from jax.experimental.pallas import tpu as pltpu
from jax.experimental.pallas import tpu_sc as plsc

_N_CODES = 8192
_DIM = 256
_TOKENS = 8192
_TT = 1024
_CT = 8192
_I32_MAX = 2**31 - 1


def _argmin_body(en_ref, iotaf_ref, z_ref, emb_ref, idx_ref):
    zn = jnp.sum(z_ref[...] ** 2, axis=1, keepdims=True)
    s2 = lax.dot_general(
        z_ref[...] * -2.0, emb_ref[...],
        dimension_numbers=(((1,), (1,)), ((), ())),
        preferred_element_type=jnp.float32,
    )
    d = (zn + en_ref[...]) + s2
    local_min = jnp.min(d, axis=1, keepdims=True)
    idx_f = jnp.min(
        jnp.where(d == local_min, iotaf_ref[...], jnp.float32(3.0e38)),
        axis=1, keepdims=True,
    )
    idx_ref[...] = idx_f.astype(jnp.int32)


def _fused_argmin(en, iotaf, z_flat, emb):
    grid = (_TOKENS // _TT,)
    return pl.pallas_call(
        _argmin_body,
        grid=grid,
        in_specs=[
            pl.BlockSpec((1, _N_CODES), lambda i: (0, 0)),
            pl.BlockSpec((1, _N_CODES), lambda i: (0, 0)),
            pl.BlockSpec((_TT, _DIM), lambda i: (i, 0)),
            pl.BlockSpec((_N_CODES, _DIM), lambda i: (0, 0)),
        ],
        out_specs=pl.BlockSpec((_TT, 1), lambda i: (i, 0)),
        out_shape=jax.ShapeDtypeStruct((_TOKENS, 1), jnp.int32),
    )(en, iotaf, z_flat, emb)


def _sc_gather(embedding, idx):
    info = plsc.get_sparse_core_info()
    nw = info.num_cores * info.num_subcores
    b_per_w = _TOKENS // nw
    mesh = plsc.VectorSubcoreMesh(core_axis_name="c", subcore_axis_name="s")

    @functools.partial(
        pl.kernel,
        mesh=mesh,
        out_type=jax.ShapeDtypeStruct((_TOKENS, _DIM), jnp.float32),
        scratch_types=[
            pltpu.VMEM((b_per_w,), jnp.int32),
            pltpu.VMEM((b_per_w, _DIM), jnp.float32),
            pltpu.SemaphoreType.DMA,
        ],
    )
    def k(table_hbm, idx_hbm, out_hbm, idx_v, rows_v, sem):
        wid = lax.axis_index("s") * info.num_cores + lax.axis_index("c")
        base = wid * b_per_w
        pltpu.sync_copy(idx_hbm.at[pl.ds(base, b_per_w)], idx_v)
        pltpu.async_copy(table_hbm.at[idx_v], rows_v, sem).wait()
        pltpu.sync_copy(rows_v, out_hbm.at[pl.ds(base, b_per_w)])

    return k(embedding, idx)


def kernel(z, embedding):
    b, c, h, w = z.shape
    z_flat = jnp.transpose(z, (0, 2, 3, 1)).reshape(-1, c)
    en = jnp.sum(embedding**2, axis=1)[None, :]
    iotaf = lax.iota(jnp.float32, _N_CODES)[None, :]
    idx2d = _fused_argmin(en, iotaf, z_flat, embedding)
    min_indices = idx2d.reshape(-1)
    z_q_flat = _sc_gather(embedding, min_indices)
    z_q_bchw = jnp.transpose(z_q_flat.reshape(b, h, w, c), (0, 3, 1, 2))
    z_q_out = z + lax.stop_gradient(z_q_bchw - z)
    loss = jnp.array(0.0, dtype=jnp.float32)
    return (z_q_out, loss, (min_indices,))
